# Initial kernel scaffold; baseline (speedup 1.0000x reference)
import jax
import jax.numpy as jnp
from jax.experimental import pallas as pl


def kernel(inputs, table):
    raise NotImplementedError("write your pallas kernel here")


# speedup vs baseline: 1.2903x; 1.2903x over previous
import functools

import jax
import jax.numpy as jnp
from jax import lax
from jax.experimental import pallas as pl
from jax.experimental.pallas import tpu as pltpu
from jax.experimental.pallas import tpu_sc as plsc

D = 32
ROW = 128
NW = 32
K = 8


@functools.lru_cache(maxsize=None)
def _build(n_idx_rows: int):
    rows_per_w = n_idx_rows // NW
    n_outer = rows_per_w // K
    mesh = plsc.VectorSubcoreMesh(core_axis_name="c", subcore_axis_name="s")

    @functools.partial(
        pl.kernel,
        mesh=mesh,
        out_type=jax.ShapeDtypeStruct((n_idx_rows, ROW, D), jnp.float32),
        scratch_types=[
            pltpu.VMEM((K, ROW), jnp.int32),
            pltpu.VMEM((K, ROW, D), jnp.float32),
            pltpu.SemaphoreType.DMA,
        ],
        compiler_params=pltpu.CompilerParams(use_tc_tiling_on_sc=False),
    )
    def gather_kernel(idx_hbm, table_hbm, out_hbm, idx_v, rows_v, sem):
        wid = lax.axis_index("s") * 2 + lax.axis_index("c")
        base = wid * rows_per_w

        def body(g, carry):
            r0 = base + g * K
            pltpu.sync_copy(idx_hbm.at[pl.ds(r0, K)], idx_v)
            handles = [
                pltpu.async_copy(table_hbm.at[idx_v.at[j]], rows_v.at[j], sem)
                for j in range(K)
            ]
            for h in handles:
                h.wait()
            pltpu.sync_copy(rows_v, out_hbm.at[pl.ds(r0, K)])
            return carry

        lax.fori_loop(0, n_outer, body, 0)

    return gather_kernel


def kernel(inputs, table):
    b, h = inputs.shape
    idx = inputs.reshape(-1, ROW)
    out = _build(idx.shape[0])(idx, table)
    return out.reshape(b, h, D)


# speedup vs baseline: 1.3078x; 1.0136x over previous
import functools

import jax
import jax.numpy as jnp
from jax import lax
from jax.experimental import pallas as pl
from jax.experimental.pallas import tpu as pltpu
from jax.experimental.pallas import tpu_sc as plsc

D = 32
ROW = 128
NW = 32
K = 10
NBUF = 2


@functools.lru_cache(maxsize=None)
def _build(n_idx_rows: int):
    n_chunks = n_idx_rows // K
    chunks_w = n_chunks // NW
    rounds = chunks_w // NBUF
    mesh = plsc.VectorSubcoreMesh(core_axis_name="c", subcore_axis_name="s")

    @functools.partial(
        pl.kernel,
        mesh=mesh,
        out_type=jax.ShapeDtypeStruct((n_idx_rows, ROW, D), jnp.float32),
        scratch_types=[
            pltpu.VMEM((NBUF, K, ROW), jnp.int32),
            pltpu.VMEM((NBUF, K, ROW, D), jnp.float32),
            pltpu.SemaphoreType.DMA,
            pltpu.SemaphoreType.DMA,
            pltpu.SemaphoreType.DMA,
            pltpu.SemaphoreType.DMA,
        ],
        compiler_params=pltpu.CompilerParams(use_tc_tiling_on_sc=False),
    )
    def gather_kernel(idx_hbm, table_hbm, out_hbm, idx_v, rows_v,
                      gs0, gs1, ws0, ws1):
        gs = [gs0, gs1]
        ws = [ws0, ws1]
        wid = lax.axis_index("s") * 2 + lax.axis_index("c")
        c0 = wid * chunks_w

        def round_(r, carry):
            cs = [c0 + r * NBUF + b for b in range(NBUF)]
            ghandles = []
            for b in range(NBUF):
                pltpu.sync_copy(idx_hbm.at[cs[b]], idx_v.at[b])
                for j in range(K):
                    ghandles.append(pltpu.async_copy(
                        table_hbm.at[idx_v.at[b].at[j]],
                        rows_v.at[b].at[j], gs[b]))
            whandles = []
            for b in range(NBUF):
                for j in range(K):
                    ghandles[b * K + j].wait()
                whandles.append(pltpu.async_copy(
                    rows_v.at[b], out_hbm.at[pl.ds(cs[b] * K, K)], ws[b]))
            for b in range(NBUF):
                whandles[b].wait()
            return carry

        lax.fori_loop(0, rounds, round_, 0)

    return gather_kernel


def kernel(inputs, table):
    b, h = inputs.shape
    idx = inputs.reshape(-1, K, ROW)
    out = _build(idx.shape[0] * K)(idx, table)
    return out.reshape(b, h, D)


# speedup vs baseline: 1.7914x; 1.3698x over previous
import functools

import jax
import jax.numpy as jnp
from jax import lax
from jax.experimental import pallas as pl
from jax.experimental.pallas import tpu as pltpu
from jax.experimental.pallas import tpu_sc as plsc

D = 32
NW = 32
NB = 16
NBUF = 2


@functools.lru_cache(maxsize=None)
def _build(batch: int, hist: int):
    rows_w = batch // NW
    chunks_w = rows_w // NB
    rounds = chunks_w // NBUF
    mesh = plsc.VectorSubcoreMesh(core_axis_name="c", subcore_axis_name="s")

    @functools.partial(
        pl.kernel,
        mesh=mesh,
        out_type=jax.ShapeDtypeStruct((batch, hist, D), jnp.float32),
        scratch_types=[
            pltpu.VMEM((NBUF, NB, hist), jnp.int32),
            pltpu.VMEM((NBUF, NB, hist, D), jnp.float32),
            pltpu.SemaphoreType.DMA,
            pltpu.SemaphoreType.DMA,
            pltpu.SemaphoreType.DMA,
            pltpu.SemaphoreType.DMA,
        ],
        compiler_params=pltpu.CompilerParams(use_tc_tiling_on_sc=False),
    )
    def gather_kernel(idx_hbm, table_hbm, out_hbm, idx_v, rows_v,
                      gs0, gs1, ws0, ws1):
        gs = [gs0, gs1]
        ws = [ws0, ws1]
        wid = lax.axis_index("s") * 2 + lax.axis_index("c")
        r0 = wid * rows_w

        def round_(r, carry):
            bs = [r0 + (r * NBUF + b) * NB for b in range(NBUF)]
            ghandles = []
            for b in range(NBUF):
                pltpu.sync_copy(idx_hbm.at[pl.ds(bs[b], NB)], idx_v.at[b])
                for j in range(NB):
                    ghandles.append(pltpu.async_copy(
                        table_hbm.at[idx_v.at[b].at[j]],
                        rows_v.at[b].at[j], gs[b]))
            whandles = []
            for b in range(NBUF):
                for j in range(NB):
                    ghandles[b * NB + j].wait()
                whandles.append(pltpu.async_copy(
                    rows_v.at[b], out_hbm.at[pl.ds(bs[b], NB)], ws[b]))
            for b in range(NBUF):
                whandles[b].wait()
            return carry

        lax.fori_loop(0, rounds, round_, 0)

    return gather_kernel


def kernel(inputs, table):
    b, h = inputs.shape
    return _build(b, h)(inputs, table)
